# Initial kernel scaffold; baseline (speedup 1.0000x reference)
import functools

import jax
import jax.numpy as jnp
from jax import lax
from jax.experimental import pallas as pl
from jax.experimental.pallas import tpu as pltpu
from jax.experimental.pallas import tpu_sc as plsc

N_IMGS, NUM_PTS, NUM_SAMPLES = 8, 4096, 32
NPTS = N_IMGS * NUM_PTS * NUM_SAMPLES
FEAT = 128
HID1 = 140
D = 144
HID2 = 70
ODIM = 3

NW = 32
NPW = NPTS // NW
KW = 64

TAIL_BLK = 4096
PTS_PER_IMG = NPTS // N_IMGS


def _table_body(e_ref, w_ref, o_ref):
    o_ref[...] = lax.dot_general(
        e_ref[...], w_ref[...], (((0,), (0,)), ((), ())),
        preferred_element_type=jnp.float32)


def _build_table(embr, w1p, blk=512):
    hw = embr.shape[1]
    return pl.pallas_call(
        _table_body,
        grid=(hw // blk,),
        in_specs=[pl.BlockSpec((FEAT, blk), lambda i: (0, i)),
                  pl.BlockSpec((FEAT, D), lambda i: (0, 0))],
        out_specs=pl.BlockSpec((blk, D), lambda i: (i, 0)),
        out_shape=jax.ShapeDtypeStruct((hw, D), jnp.float32),
    )(embr, w1p)


def _tb_body(t_ref, w_ref, b_ref, o_ref):
    o_ref[...] = lax.dot_general(
        t_ref[...], w_ref[...], (((1,), (0,)), ((), ())),
        preferred_element_type=jnp.float32) + b_ref[...]


def _build_tbias(t_feat, w1c, b1p):
    return pl.pallas_call(
        _tb_body,
        out_shape=jax.ShapeDtypeStruct((N_IMGS, D), jnp.float32),
    )(t_feat, w1c, b1p)


def _sc_body(t0, t1, xs_hbm, ys_hbm, out,
             xs_v, ys_v, idx_v, w_v, rows_v, stage_v, sem):
    wid = lax.axis_index("s") * 2 + lax.axis_index("c")
    base = wid * NPW

    @pl.loop(0, NPW // KW)
    def _win(w):
        off = base + w * KW
        pltpu.sync_copy(xs_hbm.at[pl.ds(off, KW)], xs_v)
        pltpu.sync_copy(ys_hbm.at[pl.ds(off, KW)], ys_v)

        for c in range(KW // 16):
            sl = pl.ds(c * 16, 16)
            x = xs_v[sl]
            y = ys_v[sl]
            for (side, sb) in ((128, 0), (256, 4)):
                half = (side - 1) * 0.5
                ixf = (x + 1.0) * half
                iyf = (y + 1.0) * half
                x0 = jnp.minimum(jnp.maximum(ixf.astype(jnp.int32), 0), side - 2)
                y0 = jnp.minimum(jnp.maximum(iyf.astype(jnp.int32), 0), side - 2)
                fx = ixf - x0.astype(jnp.float32)
                fy = iyf - y0.astype(jnp.float32)
                b = y0 * side + x0
                idx_v[sb + 0, sl] = b
                idx_v[sb + 1, sl] = b + 1
                idx_v[sb + 2, sl] = b + side
                idx_v[sb + 3, sl] = b + side + 1
                gx = 1.0 - fx
                gy = 1.0 - fy
                w_v[pl.ds((sb + 0) * KW + c * 16, 16)] = gx * gy
                w_v[pl.ds((sb + 1) * KW + c * 16, 16)] = fx * gy
                w_v[pl.ds((sb + 2) * KW + c * 16, 16)] = gx * fy
                w_v[pl.ds((sb + 3) * KW + c * 16, 16)] = fx * fy

        copies = []
        for j in range(4):
            copies.append(pltpu.async_copy(t0.at[idx_v.at[j]], rows_v.at[j], sem))
        for j in range(4, 8):
            copies.append(pltpu.async_copy(t1.at[idx_v.at[j]], rows_v.at[j], sem))
        for cp in copies:
            cp.wait()

        @pl.loop(0, KW)
        def _pt(p):
            wvs = [jnp.broadcast_to(w_v[j * KW + p], (16,)) for j in range(8)]
            for c in range(D // 16):
                sl = pl.ds(c * 16, 16)
                acc = wvs[0] * rows_v[0, p, sl]
                for j in range(1, 8):
                    acc = acc + wvs[j] * rows_v[j, p, sl]
                stage_v[p, sl] = acc

        pltpu.sync_copy(stage_v, out.at[pl.ds(off, KW)])


def _sc_gather(t0, t1, xs, ys):
    mesh = plsc.VectorSubcoreMesh(core_axis_name="c", subcore_axis_name="s")
    fn = pl.kernel(
        _sc_body,
        out_type=jax.ShapeDtypeStruct((NPTS, D), jnp.float32),
        mesh=mesh,
        scratch_types=[
            pltpu.VMEM((KW,), jnp.float32),
            pltpu.VMEM((KW,), jnp.float32),
            pltpu.VMEM((8, KW), jnp.int32),
            pltpu.VMEM((8 * KW,), jnp.float32),
            pltpu.VMEM((8, KW, D), jnp.float32),
            pltpu.VMEM((KW, D), jnp.float32),
            pltpu.SemaphoreType.DMA,
        ],
    )
    return fn(t0, t1, xs, ys)


def _tail_body(h_ref, tb_ref, w2_ref, b2_ref, w3_ref, b3_ref, o_ref):
    h = h_ref[...] + tb_ref[0]
    h = jnp.where(h >= 0, h, 0.01 * h)
    h2 = lax.dot_general(h, w2_ref[...], (((1,), (0,)), ((), ())),
                         preferred_element_type=jnp.float32) + b2_ref[...]
    h2 = jnp.where(h2 >= 0, h2, 0.01 * h2)
    h3 = lax.dot_general(h2, w3_ref[...], (((1,), (0,)), ((), ())),
                         preferred_element_type=jnp.float32) + b3_ref[...]
    o_ref[...] = jax.nn.sigmoid(h3)


def _tail(h1, tb3, w2p, b2r, w3, b3r):
    blocks_per_img = PTS_PER_IMG // TAIL_BLK
    return pl.pallas_call(
        _tail_body,
        grid=(NPTS // TAIL_BLK,),
        in_specs=[
            pl.BlockSpec((TAIL_BLK, D), lambda i: (i, 0)),
            pl.BlockSpec((1, 1, D), lambda i: (i // blocks_per_img, 0, 0)),
            pl.BlockSpec((D, HID2), lambda i: (0, 0)),
            pl.BlockSpec((1, HID2), lambda i: (0, 0)),
            pl.BlockSpec((HID2, ODIM), lambda i: (0, 0)),
            pl.BlockSpec((1, ODIM), lambda i: (0, 0)),
        ],
        out_specs=pl.BlockSpec((TAIL_BLK, ODIM), lambda i: (i, 0)),
        out_shape=jax.ShapeDtypeStruct((NPTS, ODIM), jnp.float32),
    )(h1, tb3, w2p, b2r, w3, b3r)


def kernel(coordinates, t_feat, emb0, emb1, W1, b1, W2, b2, W3, b3):
    coords = coordinates.reshape(-1, 2)
    xs = coords[:, 0]
    ys = coords[:, 1]

    e0r = emb0.reshape(FEAT, -1)
    e1r = emb1.reshape(FEAT, -1)
    pad = ((0, 0), (0, D - HID1))
    w1a = jnp.pad(W1[:FEAT], pad)
    w1b = jnp.pad(W1[FEAT:2 * FEAT], pad)
    w1c = jnp.pad(W1[2 * FEAT:], pad)
    b1p = jnp.pad(b1, (0, D - HID1)).reshape(1, D)

    t0 = _build_table(e0r, w1a)
    t1 = _build_table(e1r, w1b)
    tb = _build_tbias(t_feat, w1c, b1p)

    h1 = _sc_gather(t0, t1, xs, ys)

    w2p = jnp.pad(W2, ((0, D - HID1), (0, 0)))
    out = _tail(h1, tb.reshape(N_IMGS, 1, D), w2p,
                b2.reshape(1, HID2), W3, b3.reshape(1, ODIM))
    return out.reshape(N_IMGS, NUM_PTS, NUM_SAMPLES, ODIM)


# speedup vs baseline: 20.9620x; 20.9620x over previous
import dataclasses
import functools

import jax
import jax.numpy as jnp
from jax import lax
from jax.experimental import pallas as pl
from jax.experimental.pallas import tpu as pltpu
from jax.experimental.pallas import tpu_sc as plsc

N_IMGS, NUM_PTS, NUM_SAMPLES = 8, 4096, 32
NPTS = N_IMGS * NUM_PTS * NUM_SAMPLES
FEAT = 128
HID1 = 140
D = 144
HID2 = 70
ODIM = 3

NW = 32
NPW = NPTS // NW
KW = 64

TAIL_BLK = 4096
PTS_PER_IMG = NPTS // N_IMGS


def _table_body(e_ref, w_ref, o_ref):
    o_ref[...] = lax.dot_general(
        e_ref[...], w_ref[...], (((0,), (0,)), ((), ())),
        preferred_element_type=jnp.float32)


def _build_table(embr, w1p, blk=512):
    hw = embr.shape[1]
    return pl.pallas_call(
        _table_body,
        grid=(hw // blk,),
        in_specs=[pl.BlockSpec((FEAT, blk), lambda i: (0, i)),
                  pl.BlockSpec((FEAT, D), lambda i: (0, 0))],
        out_specs=pl.BlockSpec((blk, D), lambda i: (i, 0)),
        out_shape=jax.ShapeDtypeStruct((hw, D), jnp.float32),
    )(embr, w1p)


def _tb_body(t_ref, w_ref, b_ref, o_ref):
    o_ref[...] = lax.dot_general(
        t_ref[...], w_ref[...], (((1,), (0,)), ((), ())),
        preferred_element_type=jnp.float32) + b_ref[...]


def _build_tbias(t_feat, w1c, b1p):
    return pl.pallas_call(
        _tb_body,
        out_shape=jax.ShapeDtypeStruct((N_IMGS, D), jnp.float32),
    )(t_feat, w1c, b1p)


def _sc_body(t0, t1, xs_hbm, ys_hbm, out,
             xs_v, ys_v, idx_v, w_v, rows_v, stage_v, sem):
    wid = lax.axis_index("s") * 2 + lax.axis_index("c")
    base = wid * NPW

    @pl.loop(0, NPW // KW)
    def _win(w):
        off = base + w * KW
        pltpu.sync_copy(xs_hbm.at[pl.ds(off, KW)], xs_v)
        pltpu.sync_copy(ys_hbm.at[pl.ds(off, KW)], ys_v)

        for c in range(KW // 16):
            sl = pl.ds(c * 16, 16)
            x = xs_v[sl]
            y = ys_v[sl]
            for (side, sb) in ((128, 0), (256, 4)):
                half = (side - 1) * 0.5
                ixf = (x + 1.0) * half
                iyf = (y + 1.0) * half
                x0 = jnp.minimum(jnp.maximum(ixf.astype(jnp.int32), 0), side - 2)
                y0 = jnp.minimum(jnp.maximum(iyf.astype(jnp.int32), 0), side - 2)
                fx = ixf - x0.astype(jnp.float32)
                fy = iyf - y0.astype(jnp.float32)
                b = y0 * side + x0
                idx_v[sb + 0, sl] = b
                idx_v[sb + 1, sl] = b + 1
                idx_v[sb + 2, sl] = b + side
                idx_v[sb + 3, sl] = b + side + 1
                gx = 1.0 - fx
                gy = 1.0 - fy
                w_v[pl.ds((sb + 0) * KW + c * 16, 16)] = gx * gy
                w_v[pl.ds((sb + 1) * KW + c * 16, 16)] = fx * gy
                w_v[pl.ds((sb + 2) * KW + c * 16, 16)] = gx * fy
                w_v[pl.ds((sb + 3) * KW + c * 16, 16)] = fx * fy

        copies = []
        for j in range(4):
            copies.append(pltpu.async_copy(t0.at[idx_v.at[j]], rows_v.at[j], sem))
        for j in range(4, 8):
            copies.append(pltpu.async_copy(t1.at[idx_v.at[j]], rows_v.at[j], sem))
        for cp in copies:
            cp.wait()

        @pl.loop(0, KW)
        def _pt(p):
            wvs = [plsc.load_gather(
                       w_v, [jnp.broadcast_to(j * KW + p, (16,)).astype(jnp.int32)])
                   for j in range(8)]
            for c in range(D // 16):
                sl = pl.ds(c * 16, 16)
                acc = wvs[0] * rows_v[0, p, sl]
                for j in range(1, 8):
                    acc = acc + wvs[j] * rows_v[j, p, sl]
                stage_v[p, sl] = acc

        pltpu.sync_copy(stage_v, out.at[pl.ds(off, KW)])


def _sc_gather(t0, t1, xs, ys):
    mesh = plsc.VectorSubcoreMesh(core_axis_name="c", subcore_axis_name="s")
    cp = pltpu.CompilerParams(needs_layout_passes=False,
                              use_tc_tiling_on_sc=False)
    fn = pl.kernel(
        _sc_body,
        out_type=jax.ShapeDtypeStruct((NPTS, D), jnp.float32),
        mesh=mesh,
        scratch_types=[
            pltpu.VMEM((KW,), jnp.float32),
            pltpu.VMEM((KW,), jnp.float32),
            pltpu.VMEM((8, KW), jnp.int32),
            pltpu.VMEM((8 * KW,), jnp.float32),
            pltpu.VMEM((8, KW, D), jnp.float32),
            pltpu.VMEM((KW, D), jnp.float32),
            pltpu.SemaphoreType.DMA,
        ],
        compiler_params=cp,
    )
    return fn(t0, t1, xs, ys)


def _tail_body(h_ref, tb_ref, w2_ref, b2_ref, w3_ref, b3_ref, o_ref):
    h = h_ref[...] + tb_ref[0]
    h = jnp.where(h >= 0, h, 0.01 * h)
    h2 = lax.dot_general(h, w2_ref[...], (((1,), (0,)), ((), ())),
                         preferred_element_type=jnp.float32) + b2_ref[...]
    h2 = jnp.where(h2 >= 0, h2, 0.01 * h2)
    h3 = lax.dot_general(h2, w3_ref[...], (((1,), (0,)), ((), ())),
                         preferred_element_type=jnp.float32) + b3_ref[...]
    o_ref[...] = jax.nn.sigmoid(h3)


def _tail(h1, tb3, w2p, b2r, w3, b3r):
    blocks_per_img = PTS_PER_IMG // TAIL_BLK
    return pl.pallas_call(
        _tail_body,
        grid=(NPTS // TAIL_BLK,),
        in_specs=[
            pl.BlockSpec((TAIL_BLK, D), lambda i: (i, 0)),
            pl.BlockSpec((1, 1, D), lambda i: (i // blocks_per_img, 0, 0)),
            pl.BlockSpec((D, HID2), lambda i: (0, 0)),
            pl.BlockSpec((1, HID2), lambda i: (0, 0)),
            pl.BlockSpec((HID2, ODIM), lambda i: (0, 0)),
            pl.BlockSpec((1, ODIM), lambda i: (0, 0)),
        ],
        out_specs=pl.BlockSpec((TAIL_BLK, ODIM), lambda i: (i, 0)),
        out_shape=jax.ShapeDtypeStruct((NPTS, ODIM), jnp.float32),
    )(h1, tb3, w2p, b2r, w3, b3r)


def kernel(coordinates, t_feat, emb0, emb1, W1, b1, W2, b2, W3, b3):
    coords = coordinates.reshape(-1, 2)
    xs = coords[:, 0]
    ys = coords[:, 1]

    e0r = emb0.reshape(FEAT, -1)
    e1r = emb1.reshape(FEAT, -1)
    pad = ((0, 0), (0, D - HID1))
    w1a = jnp.pad(W1[:FEAT], pad)
    w1b = jnp.pad(W1[FEAT:2 * FEAT], pad)
    w1c = jnp.pad(W1[2 * FEAT:], pad)
    b1p = jnp.pad(b1, (0, D - HID1)).reshape(1, D)

    t0 = _build_table(e0r, w1a)
    t1 = _build_table(e1r, w1b)
    tb = _build_tbias(t_feat, w1c, b1p)

    h1 = _sc_gather(t0, t1, xs, ys)

    w2p = jnp.pad(W2, ((0, D - HID1), (0, 0)))
    out = _tail(h1, tb.reshape(N_IMGS, 1, D), w2p,
                b2.reshape(1, HID2), W3, b3.reshape(1, ODIM))
    return out.reshape(N_IMGS, NUM_PTS, NUM_SAMPLES, ODIM)


# speedup vs baseline: 37.1397x; 1.7718x over previous
import dataclasses
import functools

import jax
import jax.numpy as jnp
import numpy as np
from jax import lax
from jax.experimental import pallas as pl
from jax.experimental.pallas import tpu as pltpu
from jax.experimental.pallas import tpu_sc as plsc

N_IMGS, NUM_PTS, NUM_SAMPLES = 8, 4096, 32
NPTS = N_IMGS * NUM_PTS * NUM_SAMPLES
FEAT = 128
HID1 = 140
D = 160
HID2 = 70
ODIM = 3

NW = 32
NCHUNK = 1
CPTS = NPTS // NCHUNK
NPW = CPTS // NW
KW = 64
SUPER = 4096
NSB = NPW // SUPER
WPS = SUPER // KW

TAIL_BLK = 4096
PTS_PER_IMG = NPTS // N_IMGS

_COL_PERM = np.empty((D,), dtype=np.int32)
for _g in range(D // 32):
    for _j in range(16):
        _COL_PERM[32 * _g + 2 * _j] = 32 * _g + _j
        _COL_PERM[32 * _g + 2 * _j + 1] = 32 * _g + 16 + _j


def _table_body(e_ref, w_ref, o_ref):
    o_ref[...] = lax.dot_general(
        e_ref[...], w_ref[...], (((0,), (0,)), ((), ())),
        preferred_element_type=jnp.float32).astype(jnp.bfloat16)


def _build_table(embr, w1p, blk=512):
    hw = embr.shape[1]
    return pl.pallas_call(
        _table_body,
        grid=(hw // blk,),
        in_specs=[pl.BlockSpec((FEAT, blk), lambda i: (0, i)),
                  pl.BlockSpec((FEAT, D), lambda i: (0, 0))],
        out_specs=pl.BlockSpec((blk, D), lambda i: (i, 0)),
        out_shape=jax.ShapeDtypeStruct((hw, D), jnp.bfloat16),
    )(embr, w1p)


def _tb_body(t_ref, w_ref, b_ref, o_ref):
    o_ref[...] = lax.dot_general(
        t_ref[...], w_ref[...], (((1,), (0,)), ((), ())),
        preferred_element_type=jnp.float32) + b_ref[...]


def _build_tbias(t_feat, w1c, b1p):
    return pl.pallas_call(
        _tb_body,
        out_shape=jax.ShapeDtypeStruct((N_IMGS, D), jnp.float32),
    )(t_feat, w1c, b1p)


def _sc_body(t0, t1, xs_hbm, ys_hbm, outa, outb,
             xc_v, yc_v, idx_v, w_v, rows_v, sta_v, stb_v,
             semg0, semg1, semo0, semo1):
    wid = lax.axis_index("s") * 2 + lax.axis_index("c")
    base = wid * NPW
    semg = (semg0, semg1)
    semo = (semo0, semo1)

    def prep(s, wloc):
        for c in range(KW // 16):
            sl = pl.ds(wloc * KW + c * 16, 16)
            x = xc_v[sl]
            y = yc_v[sl]
            for (side, sb) in ((128, 0), (256, 4)):
                half = (side - 1) * 0.5
                ixf = (x + 1.0) * half
                iyf = (y + 1.0) * half
                x0 = jnp.minimum(jnp.maximum(ixf.astype(jnp.int32), 0), side - 2)
                y0 = jnp.minimum(jnp.maximum(iyf.astype(jnp.int32), 0), side - 2)
                fx = ixf - x0.astype(jnp.float32)
                fy = iyf - y0.astype(jnp.float32)
                b = y0 * side + x0
                csl = pl.ds(c * 16, 16)
                idx_v[s, sb + 0, csl] = b
                idx_v[s, sb + 1, csl] = b + 1
                idx_v[s, sb + 2, csl] = b + side
                idx_v[s, sb + 3, csl] = b + side + 1
                gx = 1.0 - fx
                gy = 1.0 - fy
                w_v[s, pl.ds((sb + 0) * KW + c * 16, 16)] = gx * gy
                w_v[s, pl.ds((sb + 1) * KW + c * 16, 16)] = fx * gy
                w_v[s, pl.ds((sb + 2) * KW + c * 16, 16)] = gx * fy
                w_v[s, pl.ds((sb + 3) * KW + c * 16, 16)] = fx * fy
        for j in range(4):
            pltpu.async_copy(t0.at[idx_v.at[s, j]], rows_v.at[s, j], semg[s])
        for j in range(4, 8):
            pltpu.async_copy(t1.at[idx_v.at[s, j]], rows_v.at[s, j], semg[s])

    def drain_rows(s):
        for j in range(8):
            pltpu.make_async_copy(t0.at[pl.ds(0, KW)],
                                  rows_v.at[s, j], semg[s]).wait()

    def drain_out(s):
        pltpu.make_async_copy(outa.at[pl.ds(0, KW)],
                              sta_v.at[s], semo[s]).wait()
        pltpu.make_async_copy(outb.at[pl.ds(0, KW)],
                              stb_v.at[s], semo[s]).wait()

    def finish(s, off, have_prev_store):
        drain_rows(s)

        @pl.when(have_prev_store)
        def _():
            drain_out(s)

        @pl.loop(0, KW)
        def _pt(p):
            wsp = [plsc.load_gather(
                       w_v.at[s],
                       [jnp.broadcast_to(j * KW + p, (16,)).astype(jnp.int32)])
                   for j in range(8)]
            wbf = [plsc.pack(wv, wv, format=plsc.PackFormat.INTERLEAVED)
                   for wv in wsp]
            for c in range(D // 32):
                sl = pl.ds(c * 32, 32)
                acc = wbf[0] * rows_v[s, 0, p, sl]
                for j in range(1, 8):
                    acc = acc + wbf[j] * rows_v[s, j, p, sl]
                lo, hi = plsc.unpack(acc, format=plsc.PackFormat.INTERLEAVED)
                if c < 4:
                    sta_v[s, p, pl.ds(c * 32, 16)] = lo
                    sta_v[s, p, pl.ds(c * 32 + 16, 16)] = hi
                else:
                    stb_v[s, p, pl.ds(0, 16)] = lo
                    stb_v[s, p, pl.ds(16, 16)] = hi

        pltpu.async_copy(sta_v.at[s], outa.at[pl.ds(off, KW)], semo[s])
        pltpu.async_copy(stb_v.at[s], outb.at[pl.ds(off, KW)], semo[s])

    @pl.loop(0, NSB)
    def _sb(sb):
        sb_pts = base + sb * SUPER
        pltpu.sync_copy(xs_hbm.at[pl.ds(sb_pts, SUPER)], xc_v)
        pltpu.sync_copy(ys_hbm.at[pl.ds(sb_pts, SUPER)], yc_v)

        prep(0, 0)

        @pl.loop(0, WPS // 2)
        def _h(h):
            w0 = 2 * h
            prep(1, w0 + 1)
            finish(0, sb_pts + w0 * KW, h >= 1)

            @pl.when(h < WPS // 2 - 1)
            def _():
                prep(0, w0 + 2)

            finish(1, sb_pts + (w0 + 1) * KW, h >= 1)

        drain_out(0)
        drain_out(1)


def _sc_gather(t0, t1, xs, ys):
    mesh = plsc.VectorSubcoreMesh(core_axis_name="c", subcore_axis_name="s")
    cp = pltpu.CompilerParams(needs_layout_passes=False,
                              use_tc_tiling_on_sc=False)
    fn = pl.kernel(
        _sc_body,
        out_type=(jax.ShapeDtypeStruct((CPTS, 128), jnp.float32),
                  jax.ShapeDtypeStruct((CPTS, 128), jnp.float32)),
        mesh=mesh,
        scratch_types=[
            pltpu.VMEM((SUPER,), jnp.float32),
            pltpu.VMEM((SUPER,), jnp.float32),
            pltpu.VMEM((2, 8, KW), jnp.int32),
            pltpu.VMEM((2, 8 * KW), jnp.float32),
            pltpu.VMEM((2, 8, KW, D), jnp.bfloat16),
            pltpu.VMEM((2, KW, 128), jnp.float32),
            pltpu.VMEM((2, KW, 128), jnp.float32),
            pltpu.SemaphoreType.DMA,
            pltpu.SemaphoreType.DMA,
            pltpu.SemaphoreType.DMA,
            pltpu.SemaphoreType.DMA,
        ],
        compiler_params=cp,
    )
    return fn(t0, t1, xs, ys)


def _tail_body(ha_ref, hb_ref, tb_ref, w2_ref, b2_ref, w3_ref, b3_ref, o_ref):
    h = jnp.concatenate([ha_ref[...], hb_ref[:, :D - 128]], axis=1) + tb_ref[0]
    h = jnp.where(h >= 0, h, 0.01 * h)
    h2 = lax.dot_general(h.astype(jnp.bfloat16), w2_ref[...],
                         (((1,), (0,)), ((), ())),
                         preferred_element_type=jnp.float32) + b2_ref[...]
    h2 = jnp.where(h2 >= 0, h2, 0.01 * h2)
    h3 = lax.dot_general(h2.astype(jnp.bfloat16), w3_ref[...],
                         (((1,), (0,)), ((), ())),
                         preferred_element_type=jnp.float32) + b3_ref[...]
    o_ref[...] = jax.nn.sigmoid(h3)


def _tail(h1a, h1b, tb3, w2b, b2r, w3b, b3r, chunk):
    blocks_per_img = PTS_PER_IMG // TAIL_BLK
    blk0 = chunk * (CPTS // TAIL_BLK)
    return pl.pallas_call(
        _tail_body,
        grid=(CPTS // TAIL_BLK,),
        in_specs=[
            pl.BlockSpec((TAIL_BLK, 128), lambda i: (i, 0)),
            pl.BlockSpec((TAIL_BLK, 128), lambda i: (i, 0)),
            pl.BlockSpec((1, 1, D),
                         lambda i: ((blk0 + i) // blocks_per_img, 0, 0)),
            pl.BlockSpec((D, HID2), lambda i: (0, 0)),
            pl.BlockSpec((1, HID2), lambda i: (0, 0)),
            pl.BlockSpec((HID2, ODIM), lambda i: (0, 0)),
            pl.BlockSpec((1, ODIM), lambda i: (0, 0)),
        ],
        out_specs=pl.BlockSpec((TAIL_BLK, ODIM), lambda i: (i, 0)),
        out_shape=jax.ShapeDtypeStruct((CPTS, ODIM), jnp.float32),
    )(h1a, h1b, tb3, w2b, b2r, w3b, b3r)


def kernel(coordinates, t_feat, emb0, emb1, W1, b1, W2, b2, W3, b3):
    coords = coordinates.reshape(-1, 2)
    xs = coords[:, 0]
    ys = coords[:, 1]

    e0r = emb0.reshape(FEAT, -1)
    e1r = emb1.reshape(FEAT, -1)
    pad = ((0, 0), (0, D - HID1))
    perm = jnp.asarray(_COL_PERM)
    w1a = jnp.pad(W1[:FEAT], pad)[:, perm]
    w1b = jnp.pad(W1[FEAT:2 * FEAT], pad)[:, perm]
    w1c = jnp.pad(W1[2 * FEAT:], pad)
    b1p = jnp.pad(b1, (0, D - HID1)).reshape(1, D)

    t0 = _build_table(e0r, w1a)
    t1 = _build_table(e1r, w1b)
    tb = _build_tbias(t_feat, w1c, b1p)

    w2b = jnp.pad(W2, ((0, D - HID1), (0, 0))).astype(jnp.bfloat16)
    w3b = W3.astype(jnp.bfloat16)
    tb3 = tb.reshape(N_IMGS, 1, D)
    b2r = b2.reshape(1, HID2)
    b3r = b3.reshape(1, ODIM)
    outs = []
    for c in range(NCHUNK):
        sl = slice(c * CPTS, (c + 1) * CPTS)
        h1a, h1b = _sc_gather(t0, t1, xs[sl], ys[sl])
        outs.append(_tail(h1a, h1b, tb3, w2b, b2r, w3b, b3r, c))
    out = jnp.concatenate(outs, axis=0)
    return out.reshape(N_IMGS, NUM_PTS, NUM_SAMPLES, ODIM)


# speedup vs baseline: 38.6926x; 1.0418x over previous
import dataclasses
import functools

import jax
import jax.numpy as jnp
import numpy as np
from jax import lax
from jax.experimental import pallas as pl
from jax.experimental.pallas import tpu as pltpu
from jax.experimental.pallas import tpu_sc as plsc

N_IMGS, NUM_PTS, NUM_SAMPLES = 8, 4096, 32
NPTS = N_IMGS * NUM_PTS * NUM_SAMPLES
FEAT = 128
HID1 = 140
D = 160
HID2 = 70
ODIM = 3

NW = 32
NCHUNK = 2
CPTS = NPTS // NCHUNK
NPW = CPTS // NW
KW = 64
SUPER = 4096
NSB = NPW // SUPER
WPS = SUPER // KW

TAIL_BLK = 4096
PTS_PER_IMG = NPTS // N_IMGS

_COL_PERM = np.empty((D,), dtype=np.int32)
for _g in range(D // 32):
    for _j in range(16):
        _COL_PERM[32 * _g + 2 * _j] = 32 * _g + _j
        _COL_PERM[32 * _g + 2 * _j + 1] = 32 * _g + 16 + _j


def _table_body(e_ref, w_ref, o_ref):
    o_ref[...] = lax.dot_general(
        e_ref[...], w_ref[...], (((0,), (0,)), ((), ())),
        preferred_element_type=jnp.float32).astype(jnp.bfloat16)


def _build_table(embr, w1p, blk=512):
    hw = embr.shape[1]
    return pl.pallas_call(
        _table_body,
        grid=(hw // blk,),
        in_specs=[pl.BlockSpec((FEAT, blk), lambda i: (0, i)),
                  pl.BlockSpec((FEAT, D), lambda i: (0, 0))],
        out_specs=pl.BlockSpec((blk, D), lambda i: (i, 0)),
        out_shape=jax.ShapeDtypeStruct((hw, D), jnp.bfloat16),
    )(embr, w1p)


def _tb_body(t_ref, w_ref, b_ref, o_ref):
    o_ref[...] = lax.dot_general(
        t_ref[...], w_ref[...], (((1,), (0,)), ((), ())),
        preferred_element_type=jnp.float32) + b_ref[...]


def _build_tbias(t_feat, w1c, b1p):
    return pl.pallas_call(
        _tb_body,
        out_shape=jax.ShapeDtypeStruct((N_IMGS, D), jnp.float32),
    )(t_feat, w1c, b1p)


def _sc_body(t0, t1, xs_hbm, ys_hbm, outa, outb,
             xc_v, yc_v, idx_v, w_v, rows_v, sta_v, stb_v,
             semg0, semg1, semo0, semo1):
    wid = lax.axis_index("s") * 2 + lax.axis_index("c")
    base = wid * NPW
    semg = (semg0, semg1)
    semo = (semo0, semo1)

    def prep(s, wloc):
        for c in range(KW // 16):
            sl = pl.ds(wloc * KW + c * 16, 16)
            x = xc_v[sl]
            y = yc_v[sl]
            for (side, sb) in ((128, 0), (256, 4)):
                half = (side - 1) * 0.5
                ixf = (x + 1.0) * half
                iyf = (y + 1.0) * half
                x0 = jnp.minimum(jnp.maximum(ixf.astype(jnp.int32), 0), side - 2)
                y0 = jnp.minimum(jnp.maximum(iyf.astype(jnp.int32), 0), side - 2)
                fx = ixf - x0.astype(jnp.float32)
                fy = iyf - y0.astype(jnp.float32)
                b = y0 * side + x0
                csl = pl.ds(c * 16, 16)
                idx_v[s, sb + 0, csl] = b
                idx_v[s, sb + 1, csl] = b + 1
                idx_v[s, sb + 2, csl] = b + side
                idx_v[s, sb + 3, csl] = b + side + 1
                gx = 1.0 - fx
                gy = 1.0 - fy
                w_v[s, pl.ds((sb + 0) * KW + c * 16, 16)] = gx * gy
                w_v[s, pl.ds((sb + 1) * KW + c * 16, 16)] = fx * gy
                w_v[s, pl.ds((sb + 2) * KW + c * 16, 16)] = gx * fy
                w_v[s, pl.ds((sb + 3) * KW + c * 16, 16)] = fx * fy
        for j in range(4):
            pltpu.async_copy(t0.at[idx_v.at[s, j]], rows_v.at[s, j], semg[s])
        for j in range(4, 8):
            pltpu.async_copy(t1.at[idx_v.at[s, j]], rows_v.at[s, j], semg[s])

    def drain_rows(s):
        for j in range(8):
            pltpu.make_async_copy(t0.at[pl.ds(0, KW)],
                                  rows_v.at[s, j], semg[s]).wait()

    def drain_out(s):
        pltpu.make_async_copy(outa.at[pl.ds(0, KW)],
                              sta_v.at[s], semo[s]).wait()
        pltpu.make_async_copy(outb.at[pl.ds(0, KW)],
                              stb_v.at[s], semo[s]).wait()

    def finish(s, off, have_prev_store):
        drain_rows(s)

        @pl.when(have_prev_store)
        def _():
            drain_out(s)

        @pl.loop(0, KW)
        def _pt(p):
            wsp = [plsc.load_gather(
                       w_v.at[s],
                       [jnp.broadcast_to(j * KW + p, (16,)).astype(jnp.int32)])
                   for j in range(8)]
            wbf = [plsc.pack(wv, wv, format=plsc.PackFormat.INTERLEAVED)
                   for wv in wsp]
            for c in range(D // 32):
                sl = pl.ds(c * 32, 32)
                acc = wbf[0] * rows_v[s, 0, p, sl]
                for j in range(1, 8):
                    acc = acc + wbf[j] * rows_v[s, j, p, sl]
                lo, hi = plsc.unpack(acc, format=plsc.PackFormat.INTERLEAVED)
                if c < 4:
                    sta_v[s, p, pl.ds(c * 32, 16)] = lo
                    sta_v[s, p, pl.ds(c * 32 + 16, 16)] = hi
                else:
                    stb_v[s, p, pl.ds(0, 16)] = lo
                    stb_v[s, p, pl.ds(16, 16)] = hi

        pltpu.async_copy(sta_v.at[s], outa.at[pl.ds(off, KW)], semo[s])
        pltpu.async_copy(stb_v.at[s], outb.at[pl.ds(off, KW)], semo[s])

    @pl.loop(0, NSB)
    def _sb(sb):
        sb_pts = base + sb * SUPER
        pltpu.sync_copy(xs_hbm.at[pl.ds(sb_pts, SUPER)], xc_v)
        pltpu.sync_copy(ys_hbm.at[pl.ds(sb_pts, SUPER)], yc_v)

        prep(0, 0)

        @pl.loop(0, WPS // 2)
        def _h(h):
            w0 = 2 * h
            prep(1, w0 + 1)
            finish(0, sb_pts + w0 * KW, h >= 1)

            @pl.when(h < WPS // 2 - 1)
            def _():
                prep(0, w0 + 2)

            finish(1, sb_pts + (w0 + 1) * KW, h >= 1)

        drain_out(0)
        drain_out(1)


def _sc_gather(t0, t1, xs, ys):
    mesh = plsc.VectorSubcoreMesh(core_axis_name="c", subcore_axis_name="s")
    cp = pltpu.CompilerParams(needs_layout_passes=False,
                              use_tc_tiling_on_sc=False)
    fn = pl.kernel(
        _sc_body,
        out_type=(jax.ShapeDtypeStruct((CPTS, 128), jnp.float32),
                  jax.ShapeDtypeStruct((CPTS, 128), jnp.float32)),
        mesh=mesh,
        scratch_types=[
            pltpu.VMEM((SUPER,), jnp.float32),
            pltpu.VMEM((SUPER,), jnp.float32),
            pltpu.VMEM((2, 8, KW), jnp.int32),
            pltpu.VMEM((2, 8 * KW), jnp.float32),
            pltpu.VMEM((2, 8, KW, D), jnp.bfloat16),
            pltpu.VMEM((2, KW, 128), jnp.float32),
            pltpu.VMEM((2, KW, 128), jnp.float32),
            pltpu.SemaphoreType.DMA,
            pltpu.SemaphoreType.DMA,
            pltpu.SemaphoreType.DMA,
            pltpu.SemaphoreType.DMA,
        ],
        compiler_params=cp,
    )
    return fn(t0, t1, xs, ys)


def _tail_body(ha_ref, hb_ref, tb_ref, w2_ref, b2_ref, w3_ref, b3_ref, o_ref):
    h = jnp.concatenate([ha_ref[...], hb_ref[:, :D - 128]], axis=1) + tb_ref[0]
    h = jnp.where(h >= 0, h, 0.01 * h)
    h2 = lax.dot_general(h.astype(jnp.bfloat16), w2_ref[...],
                         (((1,), (0,)), ((), ())),
                         preferred_element_type=jnp.float32) + b2_ref[...]
    h2 = jnp.where(h2 >= 0, h2, 0.01 * h2)
    h3 = lax.dot_general(h2.astype(jnp.bfloat16), w3_ref[...],
                         (((1,), (0,)), ((), ())),
                         preferred_element_type=jnp.float32) + b3_ref[...]
    o_ref[...] = jax.nn.sigmoid(h3)


def _tail(h1a, h1b, tb3, w2b, b2r, w3b, b3r, chunk):
    blocks_per_img = PTS_PER_IMG // TAIL_BLK
    blk0 = chunk * (CPTS // TAIL_BLK)
    return pl.pallas_call(
        _tail_body,
        grid=(CPTS // TAIL_BLK,),
        in_specs=[
            pl.BlockSpec((TAIL_BLK, 128), lambda i: (i, 0)),
            pl.BlockSpec((TAIL_BLK, 128), lambda i: (i, 0)),
            pl.BlockSpec((1, 1, D),
                         lambda i: ((blk0 + i) // blocks_per_img, 0, 0)),
            pl.BlockSpec((D, HID2), lambda i: (0, 0)),
            pl.BlockSpec((1, HID2), lambda i: (0, 0)),
            pl.BlockSpec((HID2, ODIM), lambda i: (0, 0)),
            pl.BlockSpec((1, ODIM), lambda i: (0, 0)),
        ],
        out_specs=pl.BlockSpec((TAIL_BLK, ODIM), lambda i: (i, 0)),
        out_shape=jax.ShapeDtypeStruct((CPTS, ODIM), jnp.float32),
    )(h1a, h1b, tb3, w2b, b2r, w3b, b3r)


def kernel(coordinates, t_feat, emb0, emb1, W1, b1, W2, b2, W3, b3):
    coords = coordinates.reshape(-1, 2)
    xs = coords[:, 0]
    ys = coords[:, 1]

    e0r = emb0.reshape(FEAT, -1)
    e1r = emb1.reshape(FEAT, -1)
    pad = ((0, 0), (0, D - HID1))
    perm = jnp.asarray(_COL_PERM)
    w1a = jnp.pad(W1[:FEAT], pad)[:, perm]
    w1b = jnp.pad(W1[FEAT:2 * FEAT], pad)[:, perm]
    w1c = jnp.pad(W1[2 * FEAT:], pad)
    b1p = jnp.pad(b1, (0, D - HID1)).reshape(1, D)

    t0 = _build_table(e0r, w1a)
    t1 = _build_table(e1r, w1b)
    tb = _build_tbias(t_feat, w1c, b1p)

    w2b = jnp.pad(W2, ((0, D - HID1), (0, 0))).astype(jnp.bfloat16)
    w3b = W3.astype(jnp.bfloat16)
    tb3 = tb.reshape(N_IMGS, 1, D)
    b2r = b2.reshape(1, HID2)
    b3r = b3.reshape(1, ODIM)
    outs = []
    for c in range(NCHUNK):
        sl = slice(c * CPTS, (c + 1) * CPTS)
        h1a, h1b = _sc_gather(t0, t1, xs[sl], ys[sl])
        outs.append(_tail(h1a, h1b, tb3, w2b, b2r, w3b, b3r, c))
    out = jnp.concatenate(outs, axis=0)
    return out.reshape(N_IMGS, NUM_PTS, NUM_SAMPLES, ODIM)


# speedup vs baseline: 40.1776x; 1.0384x over previous
import dataclasses
import functools

import jax
import jax.numpy as jnp
import numpy as np
from jax import lax
from jax.experimental import pallas as pl
from jax.experimental.pallas import tpu as pltpu
from jax.experimental.pallas import tpu_sc as plsc

N_IMGS, NUM_PTS, NUM_SAMPLES = 8, 4096, 32
NPTS = N_IMGS * NUM_PTS * NUM_SAMPLES
FEAT = 128
HID1 = 140
D = 160
HID2 = 70
ODIM = 3

NW = 32
NCHUNK = 4
CPTS = NPTS // NCHUNK
NPW = CPTS // NW
KW = 64
SUPER = 4096
NSB = NPW // SUPER
WPS = SUPER // KW

TAIL_BLK = 4096
PTS_PER_IMG = NPTS // N_IMGS

_COL_PERM = np.empty((D,), dtype=np.int32)
for _g in range(D // 32):
    for _j in range(16):
        _COL_PERM[32 * _g + 2 * _j] = 32 * _g + _j
        _COL_PERM[32 * _g + 2 * _j + 1] = 32 * _g + 16 + _j


def _table_body(e_ref, w_ref, o_ref):
    o_ref[...] = lax.dot_general(
        e_ref[...], w_ref[...], (((0,), (0,)), ((), ())),
        preferred_element_type=jnp.float32).astype(jnp.bfloat16)


def _build_table(embr, w1p, blk=512):
    hw = embr.shape[1]
    return pl.pallas_call(
        _table_body,
        grid=(hw // blk,),
        in_specs=[pl.BlockSpec((FEAT, blk), lambda i: (0, i)),
                  pl.BlockSpec((FEAT, D), lambda i: (0, 0))],
        out_specs=pl.BlockSpec((blk, D), lambda i: (i, 0)),
        out_shape=jax.ShapeDtypeStruct((hw, D), jnp.bfloat16),
    )(embr, w1p)


def _tb_body(t_ref, w_ref, b_ref, o_ref):
    o_ref[...] = lax.dot_general(
        t_ref[...], w_ref[...], (((1,), (0,)), ((), ())),
        preferred_element_type=jnp.float32) + b_ref[...]


def _build_tbias(t_feat, w1c, b1p):
    return pl.pallas_call(
        _tb_body,
        out_shape=jax.ShapeDtypeStruct((N_IMGS, D), jnp.float32),
    )(t_feat, w1c, b1p)


def _sc_body(t0, t1, xs_hbm, ys_hbm, outa, outb,
             xc_v, yc_v, idx_v, w_v, rows_v, sta_v, stb_v,
             semg0, semg1, semo0, semo1):
    wid = lax.axis_index("s") * 2 + lax.axis_index("c")
    base = wid * NPW
    semg = (semg0, semg1)
    semo = (semo0, semo1)

    def prep(s, wloc):
        for c in range(KW // 16):
            sl = pl.ds(wloc * KW + c * 16, 16)
            x = xc_v[sl]
            y = yc_v[sl]
            for (side, sb) in ((128, 0), (256, 4)):
                half = (side - 1) * 0.5
                ixf = (x + 1.0) * half
                iyf = (y + 1.0) * half
                x0 = jnp.minimum(jnp.maximum(ixf.astype(jnp.int32), 0), side - 2)
                y0 = jnp.minimum(jnp.maximum(iyf.astype(jnp.int32), 0), side - 2)
                fx = ixf - x0.astype(jnp.float32)
                fy = iyf - y0.astype(jnp.float32)
                b = y0 * side + x0
                csl = pl.ds(c * 16, 16)
                idx_v[s, sb + 0, csl] = b
                idx_v[s, sb + 1, csl] = b + 1
                idx_v[s, sb + 2, csl] = b + side
                idx_v[s, sb + 3, csl] = b + side + 1
                gx = 1.0 - fx
                gy = 1.0 - fy
                w_v[s, pl.ds((sb + 0) * KW + c * 16, 16)] = gx * gy
                w_v[s, pl.ds((sb + 1) * KW + c * 16, 16)] = fx * gy
                w_v[s, pl.ds((sb + 2) * KW + c * 16, 16)] = gx * fy
                w_v[s, pl.ds((sb + 3) * KW + c * 16, 16)] = fx * fy
        for j in range(4):
            pltpu.async_copy(t0.at[idx_v.at[s, j]], rows_v.at[s, j], semg[s])
        for j in range(4, 8):
            pltpu.async_copy(t1.at[idx_v.at[s, j]], rows_v.at[s, j], semg[s])

    def drain_rows(s):
        for j in range(8):
            pltpu.make_async_copy(t0.at[pl.ds(0, KW)],
                                  rows_v.at[s, j], semg[s]).wait()

    def drain_out(s):
        pltpu.make_async_copy(outa.at[pl.ds(0, KW)],
                              sta_v.at[s], semo[s]).wait()
        pltpu.make_async_copy(outb.at[pl.ds(0, KW)],
                              stb_v.at[s], semo[s]).wait()

    def finish(s, off, have_prev_store):
        drain_rows(s)

        @pl.when(have_prev_store)
        def _():
            drain_out(s)

        @pl.loop(0, KW)
        def _pt(p):
            wsp = [plsc.load_gather(
                       w_v.at[s],
                       [jnp.broadcast_to(j * KW + p, (16,)).astype(jnp.int32)])
                   for j in range(8)]
            wbf = [plsc.pack(wv, wv, format=plsc.PackFormat.INTERLEAVED)
                   for wv in wsp]
            for c in range(D // 32):
                sl = pl.ds(c * 32, 32)
                acc = wbf[0] * rows_v[s, 0, p, sl]
                for j in range(1, 8):
                    acc = acc + wbf[j] * rows_v[s, j, p, sl]
                lo, hi = plsc.unpack(acc, format=plsc.PackFormat.INTERLEAVED)
                if c < 4:
                    sta_v[s, p, pl.ds(c * 32, 16)] = lo
                    sta_v[s, p, pl.ds(c * 32 + 16, 16)] = hi
                else:
                    stb_v[s, p, pl.ds(0, 16)] = lo
                    stb_v[s, p, pl.ds(16, 16)] = hi

        pltpu.async_copy(sta_v.at[s], outa.at[pl.ds(off, KW)], semo[s])
        pltpu.async_copy(stb_v.at[s], outb.at[pl.ds(off, KW)], semo[s])

    @pl.loop(0, NSB)
    def _sb(sb):
        sb_pts = base + sb * SUPER
        pltpu.sync_copy(xs_hbm.at[pl.ds(sb_pts, SUPER)], xc_v)
        pltpu.sync_copy(ys_hbm.at[pl.ds(sb_pts, SUPER)], yc_v)

        prep(0, 0)

        @pl.loop(0, WPS // 2)
        def _h(h):
            w0 = 2 * h
            prep(1, w0 + 1)
            finish(0, sb_pts + w0 * KW, h >= 1)

            @pl.when(h < WPS // 2 - 1)
            def _():
                prep(0, w0 + 2)

            finish(1, sb_pts + (w0 + 1) * KW, h >= 1)

        drain_out(0)
        drain_out(1)


def _sc_gather(t0, t1, xs, ys):
    mesh = plsc.VectorSubcoreMesh(core_axis_name="c", subcore_axis_name="s")
    cp = pltpu.CompilerParams(needs_layout_passes=False,
                              use_tc_tiling_on_sc=False)
    fn = pl.kernel(
        _sc_body,
        out_type=(jax.ShapeDtypeStruct((CPTS, 128), jnp.float32),
                  jax.ShapeDtypeStruct((CPTS, 128), jnp.float32)),
        mesh=mesh,
        scratch_types=[
            pltpu.VMEM((SUPER,), jnp.float32),
            pltpu.VMEM((SUPER,), jnp.float32),
            pltpu.VMEM((2, 8, KW), jnp.int32),
            pltpu.VMEM((2, 8 * KW), jnp.float32),
            pltpu.VMEM((2, 8, KW, D), jnp.bfloat16),
            pltpu.VMEM((2, KW, 128), jnp.float32),
            pltpu.VMEM((2, KW, 128), jnp.float32),
            pltpu.SemaphoreType.DMA,
            pltpu.SemaphoreType.DMA,
            pltpu.SemaphoreType.DMA,
            pltpu.SemaphoreType.DMA,
        ],
        compiler_params=cp,
    )
    return fn(t0, t1, xs, ys)


def _tail_body(ha_ref, hb_ref, tb_ref, w2_ref, b2_ref, w3_ref, b3_ref, o_ref):
    h = jnp.concatenate([ha_ref[...], hb_ref[:, :D - 128]], axis=1) + tb_ref[0]
    h = jnp.where(h >= 0, h, 0.01 * h)
    h2 = lax.dot_general(h.astype(jnp.bfloat16), w2_ref[...],
                         (((1,), (0,)), ((), ())),
                         preferred_element_type=jnp.float32) + b2_ref[...]
    h2 = jnp.where(h2 >= 0, h2, 0.01 * h2)
    h3 = lax.dot_general(h2.astype(jnp.bfloat16), w3_ref[...],
                         (((1,), (0,)), ((), ())),
                         preferred_element_type=jnp.float32) + b3_ref[...]
    o_ref[...] = jax.nn.sigmoid(h3)


def _tail(h1a, h1b, tb3, w2b, b2r, w3b, b3r, chunk):
    blocks_per_img = PTS_PER_IMG // TAIL_BLK
    blk0 = chunk * (CPTS // TAIL_BLK)
    return pl.pallas_call(
        _tail_body,
        grid=(CPTS // TAIL_BLK,),
        in_specs=[
            pl.BlockSpec((TAIL_BLK, 128), lambda i: (i, 0)),
            pl.BlockSpec((TAIL_BLK, 128), lambda i: (i, 0)),
            pl.BlockSpec((1, 1, D),
                         lambda i: ((blk0 + i) // blocks_per_img, 0, 0)),
            pl.BlockSpec((D, HID2), lambda i: (0, 0)),
            pl.BlockSpec((1, HID2), lambda i: (0, 0)),
            pl.BlockSpec((HID2, ODIM), lambda i: (0, 0)),
            pl.BlockSpec((1, ODIM), lambda i: (0, 0)),
        ],
        out_specs=pl.BlockSpec((TAIL_BLK, ODIM), lambda i: (i, 0)),
        out_shape=jax.ShapeDtypeStruct((CPTS, ODIM), jnp.float32),
    )(h1a, h1b, tb3, w2b, b2r, w3b, b3r)


def kernel(coordinates, t_feat, emb0, emb1, W1, b1, W2, b2, W3, b3):
    coords = coordinates.reshape(-1, 2)
    xs = coords[:, 0]
    ys = coords[:, 1]

    e0r = emb0.reshape(FEAT, -1)
    e1r = emb1.reshape(FEAT, -1)
    pad = ((0, 0), (0, D - HID1))
    perm = jnp.asarray(_COL_PERM)
    w1a = jnp.pad(W1[:FEAT], pad)[:, perm]
    w1b = jnp.pad(W1[FEAT:2 * FEAT], pad)[:, perm]
    w1c = jnp.pad(W1[2 * FEAT:], pad)
    b1p = jnp.pad(b1, (0, D - HID1)).reshape(1, D)

    t0 = _build_table(e0r, w1a)
    t1 = _build_table(e1r, w1b)
    tb = _build_tbias(t_feat, w1c, b1p)

    w2b = jnp.pad(W2, ((0, D - HID1), (0, 0))).astype(jnp.bfloat16)
    w3b = W3.astype(jnp.bfloat16)
    tb3 = tb.reshape(N_IMGS, 1, D)
    b2r = b2.reshape(1, HID2)
    b3r = b3.reshape(1, ODIM)
    outs = []
    for c in range(NCHUNK):
        sl = slice(c * CPTS, (c + 1) * CPTS)
        h1a, h1b = _sc_gather(t0, t1, xs[sl], ys[sl])
        outs.append(_tail(h1a, h1b, tb3, w2b, b2r, w3b, b3r, c))
    out = jnp.concatenate(outs, axis=0)
    return out.reshape(N_IMGS, NUM_PTS, NUM_SAMPLES, ODIM)


# speedup vs baseline: 41.1734x; 1.0248x over previous
import dataclasses
import functools

import jax
import jax.numpy as jnp
import numpy as np
from jax import lax
from jax.experimental import pallas as pl
from jax.experimental.pallas import tpu as pltpu
from jax.experimental.pallas import tpu_sc as plsc

N_IMGS, NUM_PTS, NUM_SAMPLES = 8, 4096, 32
NPTS = N_IMGS * NUM_PTS * NUM_SAMPLES
FEAT = 128
HID1 = 140
D = 160
HID2 = 70
ODIM = 3

NW = 32
NCHUNK = 8
CPTS = NPTS // NCHUNK
NPW = CPTS // NW
KW = 64
SUPER = 4096
NSB = NPW // SUPER
WPS = SUPER // KW

TAIL_BLK = 4096
PTS_PER_IMG = NPTS // N_IMGS

_COL_PERM = np.empty((D,), dtype=np.int32)
for _g in range(D // 32):
    for _j in range(16):
        _COL_PERM[32 * _g + 2 * _j] = 32 * _g + _j
        _COL_PERM[32 * _g + 2 * _j + 1] = 32 * _g + 16 + _j


def _table_body(e_ref, w_ref, o_ref):
    o_ref[...] = lax.dot_general(
        e_ref[...], w_ref[...], (((0,), (0,)), ((), ())),
        preferred_element_type=jnp.float32).astype(jnp.bfloat16)


def _build_table(embr, w1p, blk=512):
    hw = embr.shape[1]
    return pl.pallas_call(
        _table_body,
        grid=(hw // blk,),
        in_specs=[pl.BlockSpec((FEAT, blk), lambda i: (0, i)),
                  pl.BlockSpec((FEAT, D), lambda i: (0, 0))],
        out_specs=pl.BlockSpec((blk, D), lambda i: (i, 0)),
        out_shape=jax.ShapeDtypeStruct((hw, D), jnp.bfloat16),
    )(embr, w1p)


def _tb_body(t_ref, w_ref, b_ref, o_ref):
    o_ref[...] = lax.dot_general(
        t_ref[...], w_ref[...], (((1,), (0,)), ((), ())),
        preferred_element_type=jnp.float32) + b_ref[...]


def _build_tbias(t_feat, w1c, b1p):
    return pl.pallas_call(
        _tb_body,
        out_shape=jax.ShapeDtypeStruct((N_IMGS, D), jnp.float32),
    )(t_feat, w1c, b1p)


def _sc_body(t0, t1, xs_hbm, ys_hbm, outa, outb,
             xc_v, yc_v, idx_v, w_v, rows_v, sta_v, stb_v,
             semg0, semg1, semo0, semo1):
    wid = lax.axis_index("s") * 2 + lax.axis_index("c")
    base = wid * NPW
    semg = (semg0, semg1)
    semo = (semo0, semo1)

    def prep(s, wloc):
        for c in range(KW // 16):
            sl = pl.ds(wloc * KW + c * 16, 16)
            x = xc_v[sl]
            y = yc_v[sl]
            for (side, sb) in ((128, 0), (256, 4)):
                half = (side - 1) * 0.5
                ixf = (x + 1.0) * half
                iyf = (y + 1.0) * half
                x0 = jnp.minimum(jnp.maximum(ixf.astype(jnp.int32), 0), side - 2)
                y0 = jnp.minimum(jnp.maximum(iyf.astype(jnp.int32), 0), side - 2)
                fx = ixf - x0.astype(jnp.float32)
                fy = iyf - y0.astype(jnp.float32)
                b = y0 * side + x0
                csl = pl.ds(c * 16, 16)
                idx_v[s, sb + 0, csl] = b
                idx_v[s, sb + 1, csl] = b + 1
                idx_v[s, sb + 2, csl] = b + side
                idx_v[s, sb + 3, csl] = b + side + 1
                gx = 1.0 - fx
                gy = 1.0 - fy
                w_v[s, pl.ds((sb + 0) * KW + c * 16, 16)] = gx * gy
                w_v[s, pl.ds((sb + 1) * KW + c * 16, 16)] = fx * gy
                w_v[s, pl.ds((sb + 2) * KW + c * 16, 16)] = gx * fy
                w_v[s, pl.ds((sb + 3) * KW + c * 16, 16)] = fx * fy
        for j in range(4):
            pltpu.async_copy(t0.at[idx_v.at[s, j]], rows_v.at[s, j], semg[s])
        for j in range(4, 8):
            pltpu.async_copy(t1.at[idx_v.at[s, j]], rows_v.at[s, j], semg[s])

    def drain_rows(s):
        for j in range(8):
            pltpu.make_async_copy(t0.at[pl.ds(0, KW)],
                                  rows_v.at[s, j], semg[s]).wait()

    def drain_out(s):
        pltpu.make_async_copy(outa.at[pl.ds(0, KW)],
                              sta_v.at[s], semo[s]).wait()
        pltpu.make_async_copy(outb.at[pl.ds(0, KW)],
                              stb_v.at[s], semo[s]).wait()

    def finish(s, off, have_prev_store):
        drain_rows(s)

        @pl.when(have_prev_store)
        def _():
            drain_out(s)

        @pl.loop(0, KW)
        def _pt(p):
            wsp = [plsc.load_gather(
                       w_v.at[s],
                       [jnp.broadcast_to(j * KW + p, (16,)).astype(jnp.int32)])
                   for j in range(8)]
            wbf = [plsc.pack(wv, wv, format=plsc.PackFormat.INTERLEAVED)
                   for wv in wsp]
            for c in range(D // 32):
                sl = pl.ds(c * 32, 32)
                acc = wbf[0] * rows_v[s, 0, p, sl]
                for j in range(1, 8):
                    acc = acc + wbf[j] * rows_v[s, j, p, sl]
                lo, hi = plsc.unpack(acc, format=plsc.PackFormat.INTERLEAVED)
                if c < 4:
                    sta_v[s, p, pl.ds(c * 32, 16)] = lo
                    sta_v[s, p, pl.ds(c * 32 + 16, 16)] = hi
                else:
                    stb_v[s, p, pl.ds(0, 16)] = lo
                    stb_v[s, p, pl.ds(16, 16)] = hi

        pltpu.async_copy(sta_v.at[s], outa.at[pl.ds(off, KW)], semo[s])
        pltpu.async_copy(stb_v.at[s], outb.at[pl.ds(off, KW)], semo[s])

    @pl.loop(0, NSB)
    def _sb(sb):
        sb_pts = base + sb * SUPER
        pltpu.sync_copy(xs_hbm.at[pl.ds(sb_pts, SUPER)], xc_v)
        pltpu.sync_copy(ys_hbm.at[pl.ds(sb_pts, SUPER)], yc_v)

        prep(0, 0)

        @pl.loop(0, WPS // 2)
        def _h(h):
            w0 = 2 * h
            prep(1, w0 + 1)
            finish(0, sb_pts + w0 * KW, h >= 1)

            @pl.when(h < WPS // 2 - 1)
            def _():
                prep(0, w0 + 2)

            finish(1, sb_pts + (w0 + 1) * KW, h >= 1)

        drain_out(0)
        drain_out(1)


def _sc_gather(t0, t1, xs, ys):
    mesh = plsc.VectorSubcoreMesh(core_axis_name="c", subcore_axis_name="s")
    cp = pltpu.CompilerParams(needs_layout_passes=False,
                              use_tc_tiling_on_sc=False)
    fn = pl.kernel(
        _sc_body,
        out_type=(jax.ShapeDtypeStruct((CPTS, 128), jnp.float32),
                  jax.ShapeDtypeStruct((CPTS, 128), jnp.float32)),
        mesh=mesh,
        scratch_types=[
            pltpu.VMEM((SUPER,), jnp.float32),
            pltpu.VMEM((SUPER,), jnp.float32),
            pltpu.VMEM((2, 8, KW), jnp.int32),
            pltpu.VMEM((2, 8 * KW), jnp.float32),
            pltpu.VMEM((2, 8, KW, D), jnp.bfloat16),
            pltpu.VMEM((2, KW, 128), jnp.float32),
            pltpu.VMEM((2, KW, 128), jnp.float32),
            pltpu.SemaphoreType.DMA,
            pltpu.SemaphoreType.DMA,
            pltpu.SemaphoreType.DMA,
            pltpu.SemaphoreType.DMA,
        ],
        compiler_params=cp,
    )
    return fn(t0, t1, xs, ys)


def _tail_body(ha_ref, hb_ref, tb_ref, w2_ref, b2_ref, w3_ref, b3_ref, o_ref):
    h = jnp.concatenate([ha_ref[...], hb_ref[:, :D - 128]], axis=1) + tb_ref[0]
    h = jnp.where(h >= 0, h, 0.01 * h)
    h2 = lax.dot_general(h.astype(jnp.bfloat16), w2_ref[...],
                         (((1,), (0,)), ((), ())),
                         preferred_element_type=jnp.float32) + b2_ref[...]
    h2 = jnp.where(h2 >= 0, h2, 0.01 * h2)
    h3 = lax.dot_general(h2.astype(jnp.bfloat16), w3_ref[...],
                         (((1,), (0,)), ((), ())),
                         preferred_element_type=jnp.float32) + b3_ref[...]
    o_ref[...] = jax.nn.sigmoid(h3)


def _tail(h1a, h1b, tb3, w2b, b2r, w3b, b3r, chunk):
    blocks_per_img = PTS_PER_IMG // TAIL_BLK
    blk0 = chunk * (CPTS // TAIL_BLK)
    return pl.pallas_call(
        _tail_body,
        grid=(CPTS // TAIL_BLK,),
        in_specs=[
            pl.BlockSpec((TAIL_BLK, 128), lambda i: (i, 0)),
            pl.BlockSpec((TAIL_BLK, 128), lambda i: (i, 0)),
            pl.BlockSpec((1, 1, D),
                         lambda i: ((blk0 + i) // blocks_per_img, 0, 0)),
            pl.BlockSpec((D, HID2), lambda i: (0, 0)),
            pl.BlockSpec((1, HID2), lambda i: (0, 0)),
            pl.BlockSpec((HID2, ODIM), lambda i: (0, 0)),
            pl.BlockSpec((1, ODIM), lambda i: (0, 0)),
        ],
        out_specs=pl.BlockSpec((TAIL_BLK, ODIM), lambda i: (i, 0)),
        out_shape=jax.ShapeDtypeStruct((CPTS, ODIM), jnp.float32),
    )(h1a, h1b, tb3, w2b, b2r, w3b, b3r)


def kernel(coordinates, t_feat, emb0, emb1, W1, b1, W2, b2, W3, b3):
    coords = coordinates.reshape(-1, 2)
    xs = coords[:, 0]
    ys = coords[:, 1]

    e0r = emb0.reshape(FEAT, -1)
    e1r = emb1.reshape(FEAT, -1)
    pad = ((0, 0), (0, D - HID1))
    perm = jnp.asarray(_COL_PERM)
    w1a = jnp.pad(W1[:FEAT], pad)[:, perm]
    w1b = jnp.pad(W1[FEAT:2 * FEAT], pad)[:, perm]
    w1c = jnp.pad(W1[2 * FEAT:], pad)
    b1p = jnp.pad(b1, (0, D - HID1)).reshape(1, D)

    t0 = _build_table(e0r, w1a)
    t1 = _build_table(e1r, w1b)
    tb = _build_tbias(t_feat, w1c, b1p)

    w2b = jnp.pad(W2, ((0, D - HID1), (0, 0))).astype(jnp.bfloat16)
    w3b = W3.astype(jnp.bfloat16)
    tb3 = tb.reshape(N_IMGS, 1, D)
    b2r = b2.reshape(1, HID2)
    b3r = b3.reshape(1, ODIM)
    outs = []
    for c in range(NCHUNK):
        sl = slice(c * CPTS, (c + 1) * CPTS)
        h1a, h1b = _sc_gather(t0, t1, xs[sl], ys[sl])
        outs.append(_tail(h1a, h1b, tb3, w2b, b2r, w3b, b3r, c))
    out = jnp.concatenate(outs, axis=0)
    return out.reshape(N_IMGS, NUM_PTS, NUM_SAMPLES, ODIM)
